# Initial kernel scaffold; baseline (speedup 1.0000x reference)
import jax
import jax.numpy as jnp
from jax.experimental import pallas as pl


def kernel(x, ei, ew, W1, b1, Wm, bm):
    raise NotImplementedError("write your pallas kernel here")


# speedup vs baseline: 20.1245x; 20.1245x over previous
import dataclasses

import jax
import jax.numpy as jnp
from jax import lax
from jax.experimental import pallas as pl
from jax.experimental.pallas import tpu as pltpu
from jax.experimental.pallas import tpu_sc as plsc

_NC = 2
_NS = 16
_NB = 4


def _sc_params():
    cp = pltpu.CompilerParams()
    fields = pltpu.CompilerParams.__dataclass_fields__
    if "needs_layout_passes" in fields:
        cp = dataclasses.replace(cp, needs_layout_passes=False)
    if "use_tc_tiling_on_sc" in fields:
        cp = dataclasses.replace(cp, use_tc_tiling_on_sc=False)
    return cp


def _zero_vmem(ref, nrows, d):
    @pl.loop(0, nrows)
    def _(r):
        for l in range(d // 16):
            ref.at[r, pl.ds(l * 16, 16)][...] = jnp.zeros((16,), jnp.float32)


def _zero_acc_slice(acc, zsrc, s, rows_per_tile):
    zrows = zsrc.shape[0]
    for t in range(rows_per_tile // zrows):
        pltpu.sync_copy(zsrc, acc.at[pl.ds(s * rows_per_tile + t * zrows,
                                           zrows)])


def _sc_conv(table, srcr, dstr, ewr, n_acc, feature_split):
    scale = ewr is not None
    d = table.shape[-1]
    rows_e, c_w = srcr.shape
    k_per = rows_e // _NS if feature_split else rows_e // (_NC * _NS)
    rows_per_tile = n_acc // _NS
    mesh = plsc.VectorSubcoreMesh(core_axis_name="c", subcore_axis_name="s",
                                  num_cores=_NC, num_subcores=_NS)

    scratch = [
        pltpu.VMEM_SHARED((n_acc, d), jnp.float32),
        pltpu.VMEM((k_per, c_w), jnp.int32),
        pltpu.VMEM((k_per, c_w), jnp.int32),
    ]
    if scale:
        scratch.append(pltpu.VMEM((k_per, c_w), jnp.float32))
    scratch += [pltpu.VMEM((c_w, d), jnp.float32) for _ in range(_NB)]
    scratch += [pltpu.SemaphoreType.DMA for _ in range(2 * _NB)]

    def body(*refs):
        it = iter(refs)
        table_hbm = next(it)
        src_hbm = next(it)
        dst_hbm = next(it)
        ew_hbm = next(it) if scale else None
        out_hbm = next(it)
        acc = next(it)
        src_t = next(it)
        dst_t = next(it)
        ew_t = next(it) if scale else None
        rows = [next(it) for _ in range(_NB)]
        gsem = [next(it) for _ in range(_NB)]
        ssem = [next(it) for _ in range(_NB)]

        c = lax.axis_index("c")
        s = lax.axis_index("s")
        base = (s if feature_split else c * _NS + s) * k_per
        tab = table_hbm.at[c] if feature_split else table_hbm
        pltpu.sync_copy(src_hbm.at[pl.ds(base, k_per)], src_t)
        pltpu.sync_copy(dst_hbm.at[pl.ds(base, k_per)], dst_t)
        if scale:
            pltpu.sync_copy(ew_hbm.at[pl.ds(base, k_per)], ew_t)
        _zero_vmem(rows[0], c_w, d)
        _zero_acc_slice(acc, rows[0], s, rows_per_tile)
        plsc.subcore_barrier()

        def fire_gather(k, b):
            pltpu.async_copy(tab.at[src_t.at[k]], rows[b], gsem[b])

        def wait_gather(b):
            pltpu.make_async_copy(tab.at[src_t.at[0]], rows[b],
                                  gsem[b]).wait()

        def fire_scatter(k, b):
            pltpu.async_copy(rows[b], acc.at[dst_t.at[k]], ssem[b], add=True)

        def wait_scatter(b):
            pltpu.make_async_copy(rows[b], acc.at[dst_t.at[0]],
                                  ssem[b]).wait()

        def process(k, b):
            wait_gather(b)
            if scale:
                @pl.loop(0, c_w // 16)
                def _(g):
                    wv = ew_t.at[k, pl.ds(g * 16, 16)][...]
                    for i in range(16):
                        w = wv[i]
                        for l in range(d // 16):
                            sl = pl.ds(l * 16, 16)
                            rows[b].at[g * 16 + i, sl][...] = (
                                rows[b].at[g * 16 + i, sl][...] * w)
            fire_scatter(k, b)

        fire_gather(0, 0)
        fire_gather(1, 1)
        nj = (k_per + _NB - 1) // _NB

        @pl.loop(0, nj)
        def _(j):
            for b in range(_NB):
                k = j * _NB + b
                b2 = (b + 2) % _NB

                @pl.when(jnp.logical_and(k - 2 >= 0, k - 2 < k_per))
                def _(b2=b2):
                    wait_scatter(b2)

                @pl.when(k + 2 < k_per)
                def _(k=k, b2=b2):
                    fire_gather(k + 2, b2)

                @pl.when(k < k_per)
                def _(k=k, b=b):
                    process(k, b)

        if k_per >= 2:
            wait_scatter((k_per - 2) % _NB)
        wait_scatter((k_per - 1) % _NB)
        plsc.subcore_barrier()
        pltpu.sync_copy(acc.at[pl.ds(s * rows_per_tile, rows_per_tile)],
                        out_hbm.at[c, pl.ds(s * rows_per_tile, rows_per_tile)])

    args = [table, srcr, dstr] + ([ewr] if scale else [])
    return pl.kernel(
        body,
        out_type=jax.ShapeDtypeStruct((_NC, n_acc, d), jnp.float32),
        mesh=mesh,
        scratch_types=scratch,
        compiler_params=_sc_params(),
    )(*args)


def _sc_degrees(dstr, ewr, n_acc):
    dd = 16
    rows_e, c_w = dstr.shape
    k_per = rows_e // (_NC * _NS)
    rows_per_tile = n_acc // _NS
    mesh = plsc.VectorSubcoreMesh(core_axis_name="c", subcore_axis_name="s",
                                  num_cores=_NC, num_subcores=_NS)

    scratch = [
        pltpu.VMEM_SHARED((n_acc, dd), jnp.float32),
        pltpu.VMEM((k_per, c_w), jnp.int32),
        pltpu.VMEM((k_per, c_w), jnp.float32),
    ]
    scratch += [pltpu.VMEM((c_w, dd), jnp.float32) for _ in range(_NB)]
    scratch += [pltpu.SemaphoreType.DMA for _ in range(_NB)]

    def body(dst_hbm, ew_hbm, out_hbm, *refs):
        acc, dst_t, ew_t = refs[:3]
        pairs = list(refs[3:3 + _NB])
        ssem = list(refs[3 + _NB:])

        c = lax.axis_index("c")
        s = lax.axis_index("s")
        base = (c * _NS + s) * k_per
        pltpu.sync_copy(dst_hbm.at[pl.ds(base, k_per)], dst_t)
        pltpu.sync_copy(ew_hbm.at[pl.ds(base, k_per)], ew_t)
        _zero_vmem(pairs[0], c_w, dd)
        _zero_acc_slice(acc, pairs[0], s, rows_per_tile)
        iota = lax.iota(jnp.int32, 16)
        ones_i = jnp.ones((16,), jnp.int32)
        ones_f = jnp.ones((16,), jnp.float32)
        zeros_i = jnp.zeros((16,), jnp.int32)
        for b in range(1, _NB):
            _zero_vmem(pairs[b], c_w, dd)
        for b in range(_NB):
            for g in range(c_w // 16):
                plsc.store_scatter(pairs[b], [iota + g * 16, ones_i], ones_f)
        plsc.subcore_barrier()

        def build(k, b):
            for g in range(c_w // 16):
                vals = ew_t.at[k, pl.ds(g * 16, 16)][...]
                plsc.store_scatter(pairs[b], [iota + g * 16, zeros_i], vals)

        def fire(k, b):
            pltpu.async_copy(pairs[b], acc.at[dst_t.at[k]], ssem[b], add=True)

        def wait(b):
            pltpu.make_async_copy(pairs[b], acc.at[dst_t.at[0]],
                                  ssem[b]).wait()

        nj = (k_per + _NB - 1) // _NB

        @pl.loop(0, nj)
        def _(j):
            for b in range(_NB):
                k = j * _NB + b

                @pl.when(k < k_per)
                def _(k=k, b=b):
                    @pl.when(j > 0)
                    def _():
                        wait(b)
                    build(k, b)
                    fire(k, b)

        for b in range(min(_NB, k_per)):
            wait(b)
        plsc.subcore_barrier()
        pltpu.sync_copy(acc.at[pl.ds(s * rows_per_tile, rows_per_tile)],
                        out_hbm.at[c, pl.ds(s * rows_per_tile, rows_per_tile)])

    return pl.kernel(
        body,
        out_type=jax.ShapeDtypeStruct((_NC, n_acc, dd), jnp.float32),
        mesh=mesh,
        scratch_types=scratch,
        compiler_params=_sc_params(),
    )(dstr, ewr)


def _tc_prep(x, w1, dacc, n, r):
    xd, hd = w1.shape
    h2 = hd // 2

    def body(d_ref, x_ref, w_ref, hs_ref, dv_ref):
        deg = d_ref[0] + d_ref[1] + 1.0
        dinv = lax.rsqrt(deg)
        h = jnp.dot(x_ref[...], w_ref[...], preferred_element_type=jnp.float32)
        hs = h * dinv[:, 0:1]
        hs_ref[0] = hs[:, :h2]
        hs_ref[1] = hs[:, h2:]
        dv_ref[...] = dinv

    return pl.pallas_call(
        body,
        grid=(n // r,),
        in_specs=[
            pl.BlockSpec((2, r, 2), lambda i: (0, i, 0)),
            pl.BlockSpec((r, xd), lambda i: (i, 0)),
            pl.BlockSpec((xd, hd), lambda i: (0, 0)),
        ],
        out_specs=[
            pl.BlockSpec((2, r, h2), lambda i: (0, i, 0)),
            pl.BlockSpec((r, 2), lambda i: (i, 0)),
        ],
        out_shape=[
            jax.ShapeDtypeStruct((2, n, h2), jnp.float32),
            jax.ShapeDtypeStruct((n, 2), jnp.float32),
        ],
    )(dacc[:, :, 0:2], x, w1)


def _tc_mid(p1, hs1s, dv, b1, wm, n, r):
    hd = wm.shape[0]
    zd = wm.shape[1]
    h2 = hd // 2

    def body(p_ref, hs_ref, dv_ref, b_ref, w_ref, out_ref):
        d1 = dv_ref[...][:, 0:1]
        t_lo = (p_ref[0] + hs_ref[0]) * d1 + b_ref[...][:, :h2]
        t_hi = (p_ref[1] + hs_ref[1]) * d1 + b_ref[...][:, h2:]
        h_lo = jnp.maximum(t_lo, 0.0)
        h_hi = jnp.maximum(t_hi, 0.0)
        out = (jnp.dot(h_lo, w_ref[...][:h2, :],
                       preferred_element_type=jnp.float32)
               + jnp.dot(h_hi, w_ref[...][h2:, :],
                         preferred_element_type=jnp.float32))
        out_ref[...] = out * dv_ref[...][:, 1:2]

    return pl.pallas_call(
        body,
        grid=(n // r,),
        in_specs=[
            pl.BlockSpec((2, r, h2), lambda i: (0, i, 0)),
            pl.BlockSpec((2, r, h2), lambda i: (0, i, 0)),
            pl.BlockSpec((r, 2), lambda i: (i, 0)),
            pl.BlockSpec((1, hd), lambda i: (0, 0)),
            pl.BlockSpec((hd, zd), lambda i: (0, 0)),
        ],
        out_specs=pl.BlockSpec((r, zd), lambda i: (i, 0)),
        out_shape=jax.ShapeDtypeStruct((n, zd), jnp.float32),
    )(p1, hs1s, dv, b1, wm)


def _tc_post(p2, hs2, dv, bm, n, r):
    zd = hs2.shape[1]

    def body(p_ref, hs_ref, dv_ref, b_ref, out_ref):
        d2 = dv_ref[...][:, 1:2]
        out_ref[...] = (p_ref[0] + p_ref[1] + hs_ref[...]) * d2 + b_ref[...]

    return pl.pallas_call(
        body,
        grid=(n // r,),
        in_specs=[
            pl.BlockSpec((2, r, zd), lambda i: (0, i, 0)),
            pl.BlockSpec((r, zd), lambda i: (i, 0)),
            pl.BlockSpec((r, 2), lambda i: (i, 0)),
            pl.BlockSpec((1, zd), lambda i: (0, 0)),
        ],
        out_specs=pl.BlockSpec((r, zd), lambda i: (i, 0)),
        out_shape=jax.ShapeDtypeStruct((n, zd), jnp.float32),
    )(p2, hs2, dv, bm)


def kernel(x, ei, ew, W1, b1, Wm, bm):
    n, xd = x.shape
    hd = W1.shape[1]
    zd = Wm.shape[1]
    e = ei.shape[1]

    stride = _NC * _NS * 128 * 8
    ep = ((e + stride - 1) // stride) * stride
    pad = ep - e
    src = ei[0]
    dst = ei[1]
    ewp = ew.astype(jnp.float32)
    if pad:
        src = jnp.concatenate([src, jnp.zeros((pad,), src.dtype)])
        dst = jnp.concatenate([dst, jnp.full((pad,), n, dst.dtype)])
        ewp = jnp.concatenate([ewp, jnp.zeros((pad,), jnp.float32)])
    src64 = src.reshape(ep // 64, 64)
    dst64 = dst.reshape(ep // 64, 64)
    ew64 = ewp.reshape(ep // 64, 64)
    src128 = src.reshape(ep // 128, 128)
    dst128 = dst.reshape(ep // 128, 128)
    ew128 = ewp.reshape(ep // 128, 128)

    n_acc = ((n + 1 + _NS * 128 - 1) // (_NS * 128)) * (_NS * 128)
    r = 2000 if n % 2000 == 0 else n

    dacc = _sc_degrees(dst128, ew128, n_acc)
    hs1s, dv = _tc_prep(x, W1, dacc, n, r)
    p1 = _sc_conv(hs1s, src64, dst64, ew64, n_acc, feature_split=True)
    hs2 = _tc_mid(p1, hs1s, dv, b1.reshape(1, hd), Wm, n, r)
    p2 = _sc_conv(hs2, src128, dst128, None, n_acc, feature_split=False)
    mean = _tc_post(p2, hs2, dv, bm.reshape(1, zd), n, r)

    z = jnp.zeros((1,), jnp.float32)
    return (mean, z, z)


# speedup vs baseline: 25.9622x; 1.2901x over previous
import dataclasses

import jax
import jax.numpy as jnp
from jax import lax
from jax.experimental import pallas as pl
from jax.experimental.pallas import tpu as pltpu
from jax.experimental.pallas import tpu_sc as plsc

_NC = 2
_NS = 16
_NB = 4


def _unpack_perm(d):
    perm = [0] * d
    for base in range(0, d, 32):
        for i in range(16):
            perm[base + 2 * i] = base + i
            perm[base + 2 * i + 1] = base + 16 + i
    return perm


def _sc_params():
    cp = pltpu.CompilerParams()
    fields = pltpu.CompilerParams.__dataclass_fields__
    if "needs_layout_passes" in fields:
        cp = dataclasses.replace(cp, needs_layout_passes=False)
    if "use_tc_tiling_on_sc" in fields:
        cp = dataclasses.replace(cp, use_tc_tiling_on_sc=False)
    return cp


def _zero_vmem(ref, nrows, d):
    @pl.loop(0, nrows)
    def _(r):
        for l in range(d // 16):
            ref.at[r, pl.ds(l * 16, 16)][...] = jnp.zeros((16,), jnp.float32)


def _zero_acc_slice(acc, zsrc, s, rows_per_tile):
    zrows = zsrc.shape[0]
    for t in range(rows_per_tile // zrows):
        pltpu.sync_copy(zsrc, acc.at[pl.ds(s * rows_per_tile + t * zrows,
                                           zrows)])


def _sc_conv(table, srcr, dstr, ewr, n_acc, feature_split):
    scale = ewr is not None
    d = table.shape[-1]
    rows_e, c_w = srcr.shape
    k_per = rows_e // _NS if feature_split else rows_e // (_NC * _NS)
    rows_per_tile = n_acc // _NS
    mesh = plsc.VectorSubcoreMesh(core_axis_name="c", subcore_axis_name="s",
                                  num_cores=_NC, num_subcores=_NS)

    scratch = [
        pltpu.VMEM_SHARED((n_acc, d), jnp.float32),
        pltpu.VMEM((k_per, c_w), jnp.int32),
        pltpu.VMEM((k_per, c_w), jnp.int32),
    ]
    if scale:
        scratch.append(pltpu.VMEM((k_per, c_w), jnp.float32))
    scratch += [pltpu.VMEM((c_w, d), jnp.bfloat16) for _ in range(_NB)]
    scratch += [pltpu.VMEM((c_w, d), jnp.float32) for _ in range(_NB)]
    scratch += [pltpu.SemaphoreType.DMA for _ in range(2 * _NB)]

    def body(*refs):
        it = iter(refs)
        table_hbm = next(it)
        src_hbm = next(it)
        dst_hbm = next(it)
        ew_hbm = next(it) if scale else None
        out_hbm = next(it)
        acc = next(it)
        src_t = next(it)
        dst_t = next(it)
        ew_t = next(it) if scale else None
        rows_bf = [next(it) for _ in range(_NB)]
        rows = [next(it) for _ in range(_NB)]
        gsem = [next(it) for _ in range(_NB)]
        ssem = [next(it) for _ in range(_NB)]

        c = lax.axis_index("c")
        s = lax.axis_index("s")
        base = (s if feature_split else c * _NS + s) * k_per
        tab = table_hbm.at[c] if feature_split else table_hbm
        pltpu.sync_copy(src_hbm.at[pl.ds(base, k_per)], src_t)
        pltpu.sync_copy(dst_hbm.at[pl.ds(base, k_per)], dst_t)
        if scale:
            pltpu.sync_copy(ew_hbm.at[pl.ds(base, k_per)], ew_t)
        _zero_vmem(rows[0], c_w, d)
        _zero_acc_slice(acc, rows[0], s, rows_per_tile)
        plsc.subcore_barrier()

        def fire_gather(k, b):
            pltpu.async_copy(tab.at[src_t.at[k]], rows_bf[b], gsem[b])

        def wait_gather(b):
            pltpu.make_async_copy(tab.at[src_t.at[0]], rows_bf[b],
                                  gsem[b]).wait()

        def fire_scatter(k, b):
            pltpu.async_copy(rows[b], acc.at[dst_t.at[k]], ssem[b], add=True)

        def wait_scatter(b):
            pltpu.make_async_copy(rows[b], acc.at[dst_t.at[0]],
                                  ssem[b]).wait()

        def process(k, b):
            wait_gather(b)

            @pl.loop(0, c_w // 16)
            def _(g):
                if scale:
                    wv = ew_t.at[k, pl.ds(g * 16, 16)][...]
                for i in range(16):
                    row = g * 16 + i
                    if scale:
                        w = wv[i]
                    for l in range(d // 32):
                        ab = rows_bf[b].at[row, pl.ds(l * 32, 32)][...]
                        lo, hi = plsc.unpack(ab,
                                             format=plsc.PackFormat.INTERLEAVED)
                        if scale:
                            lo = lo * w
                            hi = hi * w
                        rows[b].at[row, pl.ds(l * 32, 16)][...] = lo
                        rows[b].at[row, pl.ds(l * 32 + 16, 16)][...] = hi
            fire_scatter(k, b)

        fire_gather(0, 0)
        fire_gather(1, 1)
        nj = (k_per + _NB - 1) // _NB

        @pl.loop(0, nj)
        def _(j):
            for b in range(_NB):
                k = j * _NB + b
                b2 = (b + 2) % _NB

                @pl.when(jnp.logical_and(k - 2 >= 0, k - 2 < k_per))
                def _(b2=b2):
                    wait_scatter(b2)

                @pl.when(k + 2 < k_per)
                def _(k=k, b2=b2):
                    fire_gather(k + 2, b2)

                @pl.when(k < k_per)
                def _(k=k, b=b):
                    process(k, b)

        if k_per >= 2:
            wait_scatter((k_per - 2) % _NB)
        wait_scatter((k_per - 1) % _NB)
        plsc.subcore_barrier()
        pltpu.sync_copy(acc.at[pl.ds(s * rows_per_tile, rows_per_tile)],
                        out_hbm.at[c, pl.ds(s * rows_per_tile, rows_per_tile)])

    args = [table, srcr, dstr] + ([ewr] if scale else [])
    return pl.kernel(
        body,
        out_type=jax.ShapeDtypeStruct((_NC, n_acc, d), jnp.float32),
        mesh=mesh,
        scratch_types=scratch,
        compiler_params=_sc_params(),
    )(*args)


def _sc_degrees(dstr, ewr, n_acc):
    dd = 16
    rows_e, c_w = dstr.shape
    k_per = rows_e // (_NC * _NS)
    rows_per_tile = n_acc // _NS
    mesh = plsc.VectorSubcoreMesh(core_axis_name="c", subcore_axis_name="s",
                                  num_cores=_NC, num_subcores=_NS)

    scratch = [
        pltpu.VMEM_SHARED((n_acc, dd), jnp.float32),
        pltpu.VMEM((k_per, c_w), jnp.int32),
        pltpu.VMEM((k_per, c_w), jnp.float32),
    ]
    scratch += [pltpu.VMEM((c_w, dd), jnp.float32) for _ in range(_NB)]
    scratch += [pltpu.SemaphoreType.DMA for _ in range(_NB)]

    def body(dst_hbm, ew_hbm, out_hbm, *refs):
        acc, dst_t, ew_t = refs[:3]
        pairs = list(refs[3:3 + _NB])
        ssem = list(refs[3 + _NB:])

        c = lax.axis_index("c")
        s = lax.axis_index("s")
        base = (c * _NS + s) * k_per
        pltpu.sync_copy(dst_hbm.at[pl.ds(base, k_per)], dst_t)
        pltpu.sync_copy(ew_hbm.at[pl.ds(base, k_per)], ew_t)
        _zero_vmem(pairs[0], c_w, dd)
        _zero_acc_slice(acc, pairs[0], s, rows_per_tile)
        iota = lax.iota(jnp.int32, 16)
        ones_i = jnp.ones((16,), jnp.int32)
        ones_f = jnp.ones((16,), jnp.float32)
        zeros_i = jnp.zeros((16,), jnp.int32)
        for b in range(1, _NB):
            _zero_vmem(pairs[b], c_w, dd)
        for b in range(_NB):
            for g in range(c_w // 16):
                plsc.store_scatter(pairs[b], [iota + g * 16, ones_i], ones_f)
        plsc.subcore_barrier()

        def build(k, b):
            for g in range(c_w // 16):
                vals = ew_t.at[k, pl.ds(g * 16, 16)][...]
                plsc.store_scatter(pairs[b], [iota + g * 16, zeros_i], vals)

        def fire(k, b):
            pltpu.async_copy(pairs[b], acc.at[dst_t.at[k]], ssem[b], add=True)

        def wait(b):
            pltpu.make_async_copy(pairs[b], acc.at[dst_t.at[0]],
                                  ssem[b]).wait()

        nj = (k_per + _NB - 1) // _NB

        @pl.loop(0, nj)
        def _(j):
            for b in range(_NB):
                k = j * _NB + b

                @pl.when(k < k_per)
                def _(k=k, b=b):
                    @pl.when(j > 0)
                    def _():
                        wait(b)
                    build(k, b)
                    fire(k, b)

        for b in range(min(_NB, k_per)):
            wait(b)
        plsc.subcore_barrier()
        pltpu.sync_copy(acc.at[pl.ds(s * rows_per_tile, rows_per_tile)],
                        out_hbm.at[c, pl.ds(s * rows_per_tile, rows_per_tile)])

    return pl.kernel(
        body,
        out_type=jax.ShapeDtypeStruct((_NC, n_acc, dd), jnp.float32),
        mesh=mesh,
        scratch_types=scratch,
        compiler_params=_sc_params(),
    )(dstr, ewr)


def _tc_prep(x, w1, dacc, n, r):
    xd, hd = w1.shape
    h2 = hd // 2

    def body(d_ref, x_ref, w_ref, hs_ref, dv_ref):
        deg = d_ref[0] + d_ref[1] + 1.0
        dinv = lax.rsqrt(deg)
        h = jnp.dot(x_ref[...], w_ref[...], preferred_element_type=jnp.float32)
        hs = h * dinv[:, 0:1]
        hs_ref[0] = hs[:, :h2]
        hs_ref[1] = hs[:, h2:]
        dv_ref[...] = dinv

    return pl.pallas_call(
        body,
        grid=(n // r,),
        in_specs=[
            pl.BlockSpec((2, r, 2), lambda i: (0, i, 0)),
            pl.BlockSpec((r, xd), lambda i: (i, 0)),
            pl.BlockSpec((xd, hd), lambda i: (0, 0)),
        ],
        out_specs=[
            pl.BlockSpec((2, r, h2), lambda i: (0, i, 0)),
            pl.BlockSpec((r, 2), lambda i: (i, 0)),
        ],
        out_shape=[
            jax.ShapeDtypeStruct((2, n, h2), jnp.float32),
            jax.ShapeDtypeStruct((n, 2), jnp.float32),
        ],
    )(dacc[:, :, 0:2], x, w1)


def _tc_mid(p1, hs1s, dv, b1, wm, n, r):
    hd = wm.shape[0]
    zd = wm.shape[1]
    h2 = hd // 2

    def body(p_ref, hs_ref, dv_ref, b_ref, w_ref, out_ref):
        d1 = dv_ref[...][:, 0:1]
        t_lo = (p_ref[0] + hs_ref[0]) * d1 + b_ref[...][:, :h2]
        t_hi = (p_ref[1] + hs_ref[1]) * d1 + b_ref[...][:, h2:]
        h_lo = jnp.maximum(t_lo, 0.0)
        h_hi = jnp.maximum(t_hi, 0.0)
        out = (jnp.dot(h_lo, w_ref[...][:h2, :],
                       preferred_element_type=jnp.float32)
               + jnp.dot(h_hi, w_ref[...][h2:, :],
                         preferred_element_type=jnp.float32))
        out_ref[...] = out * dv_ref[...][:, 1:2]

    return pl.pallas_call(
        body,
        grid=(n // r,),
        in_specs=[
            pl.BlockSpec((2, r, h2), lambda i: (0, i, 0)),
            pl.BlockSpec((2, r, h2), lambda i: (0, i, 0)),
            pl.BlockSpec((r, 2), lambda i: (i, 0)),
            pl.BlockSpec((1, hd), lambda i: (0, 0)),
            pl.BlockSpec((hd, zd), lambda i: (0, 0)),
        ],
        out_specs=pl.BlockSpec((r, zd), lambda i: (i, 0)),
        out_shape=jax.ShapeDtypeStruct((n, zd), jnp.float32),
    )(p1, hs1s, dv, b1, wm)


def _tc_post(p2, hs2, dv, bm, n, r):
    zd = hs2.shape[1]

    def body(p_ref, hs_ref, dv_ref, b_ref, out_ref):
        d2 = dv_ref[...][:, 1:2]
        out_ref[...] = (p_ref[0] + p_ref[1] + hs_ref[...]) * d2 + b_ref[...]

    return pl.pallas_call(
        body,
        grid=(n // r,),
        in_specs=[
            pl.BlockSpec((2, r, zd), lambda i: (0, i, 0)),
            pl.BlockSpec((r, zd), lambda i: (i, 0)),
            pl.BlockSpec((r, 2), lambda i: (i, 0)),
            pl.BlockSpec((1, zd), lambda i: (0, 0)),
        ],
        out_specs=pl.BlockSpec((r, zd), lambda i: (i, 0)),
        out_shape=jax.ShapeDtypeStruct((n, zd), jnp.float32),
    )(p2, hs2, dv, bm)


def kernel(x, ei, ew, W1, b1, Wm, bm):
    n, xd = x.shape
    hd = W1.shape[1]
    zd = Wm.shape[1]
    e = ei.shape[1]

    stride = _NC * _NS * 128 * 8
    ep = ((e + stride - 1) // stride) * stride
    pad = ep - e
    src = ei[0]
    dst = ei[1]
    ewp = ew.astype(jnp.float32)
    if pad:
        src = jnp.concatenate([src, jnp.zeros((pad,), src.dtype)])
        dst = jnp.concatenate([dst, jnp.full((pad,), n, dst.dtype)])
        ewp = jnp.concatenate([ewp, jnp.zeros((pad,), jnp.float32)])
    src64 = src.reshape(ep // 64, 64)
    dst64 = dst.reshape(ep // 64, 64)
    ew64 = ewp.reshape(ep // 64, 64)
    src128 = src.reshape(ep // 128, 128)
    dst128 = dst.reshape(ep // 128, 128)
    ew128 = ewp.reshape(ep // 128, 128)

    n_acc = ((n + 1 + _NS * 128 - 1) // (_NS * 128)) * (_NS * 128)
    r = 2000 if n % 2000 == 0 else n

    dacc = _sc_degrees(dst128, ew128, n_acc)
    hs1s, dv = _tc_prep(x, W1, dacc, n, r)
    tab1 = hs1s[:, :, jnp.array(_unpack_perm(hd // 2))].astype(jnp.bfloat16)
    p1 = _sc_conv(tab1, src64, dst64, ew64, n_acc, feature_split=True)
    hs2 = _tc_mid(p1, hs1s, dv, b1.reshape(1, hd), Wm, n, r)
    tab2 = hs2[:, jnp.array(_unpack_perm(zd))].astype(jnp.bfloat16)
    p2 = _sc_conv(tab2, src128, dst128, None, n_acc, feature_split=False)
    mean = _tc_post(p2, hs2, dv, bm.reshape(1, zd), n, r)

    z = jnp.zeros((1,), jnp.float32)
    return (mean, z, z)


# speedup vs baseline: 28.7368x; 1.1069x over previous
import dataclasses

import jax
import jax.numpy as jnp
from jax import lax
from jax.experimental import pallas as pl
from jax.experimental.pallas import tpu as pltpu
from jax.experimental.pallas import tpu_sc as plsc

_NC = 2
_NS = 16
_NB = 4


def _unpack_perm(d):
    perm = [0] * d
    for base in range(0, d, 32):
        for i in range(16):
            perm[base + 2 * i] = base + i
            perm[base + 2 * i + 1] = base + 16 + i
    return perm


def _sc_params():
    cp = pltpu.CompilerParams()
    fields = pltpu.CompilerParams.__dataclass_fields__
    if "needs_layout_passes" in fields:
        cp = dataclasses.replace(cp, needs_layout_passes=False)
    if "use_tc_tiling_on_sc" in fields:
        cp = dataclasses.replace(cp, use_tc_tiling_on_sc=False)
    return cp


def _zero_vmem(ref, nrows, d):
    @pl.loop(0, nrows)
    def _(r):
        for l in range(d // 16):
            ref.at[r, pl.ds(l * 16, 16)][...] = jnp.zeros((16,), jnp.float32)


def _zero_acc_slice(acc, zsrc, s, rows_per_tile):
    zrows = zsrc.shape[0]
    for t in range(rows_per_tile // zrows):
        pltpu.sync_copy(zsrc, acc.at[pl.ds(s * rows_per_tile + t * zrows,
                                           zrows)])


def _sc_conv(table, srcr, dstr, ewr, n_acc, feature_split):
    scale = ewr is not None
    d = table.shape[-1]
    rows_e, c_w = srcr.shape
    k_per = rows_e // _NS if feature_split else rows_e // (_NC * _NS)
    rows_per_tile = n_acc // _NS
    mesh = plsc.VectorSubcoreMesh(core_axis_name="c", subcore_axis_name="s",
                                  num_cores=_NC, num_subcores=_NS)

    scratch = [
        pltpu.VMEM_SHARED((n_acc, d), jnp.float32),
        pltpu.VMEM((k_per, c_w), jnp.int32),
        pltpu.VMEM((k_per, c_w), jnp.int32),
    ]
    if scale:
        scratch.append(pltpu.VMEM((k_per, c_w), jnp.float32))
    scratch += [pltpu.VMEM((c_w, d), jnp.bfloat16) for _ in range(_NB)]
    scratch += [pltpu.VMEM((c_w, d), jnp.float32) for _ in range(_NB)]
    scratch += [pltpu.SemaphoreType.DMA for _ in range(2 * _NB)]

    def body(*refs):
        it = iter(refs)
        table_hbm = next(it)
        src_hbm = next(it)
        dst_hbm = next(it)
        ew_hbm = next(it) if scale else None
        out_hbm = next(it)
        acc = next(it)
        src_t = next(it)
        dst_t = next(it)
        ew_t = next(it) if scale else None
        rows_bf = [next(it) for _ in range(_NB)]
        rows = [next(it) for _ in range(_NB)]
        gsem = [next(it) for _ in range(_NB)]
        ssem = [next(it) for _ in range(_NB)]

        c = lax.axis_index("c")
        s = lax.axis_index("s")
        base = (s if feature_split else c * _NS + s) * k_per
        tab = table_hbm.at[c] if feature_split else table_hbm
        pltpu.sync_copy(src_hbm.at[pl.ds(base, k_per)], src_t)
        pltpu.sync_copy(dst_hbm.at[pl.ds(base, k_per)], dst_t)
        if scale:
            pltpu.sync_copy(ew_hbm.at[pl.ds(base, k_per)], ew_t)
        _zero_vmem(rows[0], c_w, d)
        _zero_acc_slice(acc, rows[0], s, rows_per_tile)
        plsc.subcore_barrier()

        def fire_gather(k, b):
            pltpu.async_copy(tab.at[src_t.at[k]], rows_bf[b], gsem[b])

        def wait_gather(b):
            pltpu.make_async_copy(tab.at[src_t.at[0]], rows_bf[b],
                                  gsem[b]).wait()

        def fire_scatter(k, b):
            pltpu.async_copy(rows[b], acc.at[dst_t.at[k]], ssem[b], add=True)

        def wait_scatter(b):
            pltpu.make_async_copy(rows[b], acc.at[dst_t.at[0]],
                                  ssem[b]).wait()

        def process(k, b):
            wait_gather(b)

            @pl.loop(0, c_w // 16)
            def _(g):
                if scale:
                    wv = ew_t.at[k, pl.ds(g * 16, 16)][...]
                for i in range(16):
                    row = g * 16 + i
                    if scale:
                        w = wv[i]
                    for l in range(d // 32):
                        ab = rows_bf[b].at[row, pl.ds(l * 32, 32)][...]
                        lo, hi = plsc.unpack(ab,
                                             format=plsc.PackFormat.INTERLEAVED)
                        if scale:
                            lo = lo * w
                            hi = hi * w
                        rows[b].at[row, pl.ds(l * 32, 16)][...] = lo
                        rows[b].at[row, pl.ds(l * 32 + 16, 16)][...] = hi
            fire_scatter(k, b)

        fire_gather(0, 0)
        fire_gather(1, 1)
        nj = (k_per + _NB - 1) // _NB

        @pl.loop(0, nj)
        def _(j):
            for b in range(_NB):
                k = j * _NB + b
                b2 = (b + 2) % _NB

                @pl.when(jnp.logical_and(k - 2 >= 0, k - 2 < k_per))
                def _(b2=b2):
                    wait_scatter(b2)

                @pl.when(k + 2 < k_per)
                def _(k=k, b2=b2):
                    fire_gather(k + 2, b2)

                @pl.when(k < k_per)
                def _(k=k, b=b):
                    process(k, b)

        if k_per >= 2:
            wait_scatter((k_per - 2) % _NB)
        wait_scatter((k_per - 1) % _NB)
        plsc.subcore_barrier()
        pltpu.sync_copy(acc.at[pl.ds(s * rows_per_tile, rows_per_tile)],
                        out_hbm.at[c, pl.ds(s * rows_per_tile, rows_per_tile)])

    args = [table, srcr, dstr] + ([ewr] if scale else [])
    return pl.kernel(
        body,
        out_type=jax.ShapeDtypeStruct((_NC, n_acc, d), jnp.float32),
        mesh=mesh,
        scratch_types=scratch,
        compiler_params=_sc_params(),
    )(*args)


def _sc_degrees(dstr, ewr, n_acc):
    dd = 16
    rows_e, c_w = dstr.shape
    k_per = rows_e // (_NC * _NS)
    rows_per_tile = n_acc // _NS
    mesh = plsc.VectorSubcoreMesh(core_axis_name="c", subcore_axis_name="s",
                                  num_cores=_NC, num_subcores=_NS)

    scratch = [
        pltpu.VMEM_SHARED((n_acc, dd), jnp.float32),
        pltpu.VMEM((k_per, c_w), jnp.int32),
        pltpu.VMEM((k_per, c_w), jnp.float32),
    ]
    scratch += [pltpu.VMEM((c_w, dd), jnp.float32) for _ in range(_NB)]
    scratch += [pltpu.SemaphoreType.DMA for _ in range(_NB)]

    def body(dst_hbm, ew_hbm, out_hbm, *refs):
        acc, dst_t, ew_t = refs[:3]
        pairs = list(refs[3:3 + _NB])
        ssem = list(refs[3 + _NB:])

        c = lax.axis_index("c")
        s = lax.axis_index("s")
        base = (c * _NS + s) * k_per
        pltpu.sync_copy(dst_hbm.at[pl.ds(base, k_per)], dst_t)
        pltpu.sync_copy(ew_hbm.at[pl.ds(base, k_per)], ew_t)
        _zero_vmem(pairs[0], c_w, dd)
        _zero_acc_slice(acc, pairs[0], s, rows_per_tile)
        iota = lax.iota(jnp.int32, 16)
        ones_i = jnp.ones((16,), jnp.int32)
        ones_f = jnp.ones((16,), jnp.float32)
        zeros_i = jnp.zeros((16,), jnp.int32)
        for b in range(1, _NB):
            _zero_vmem(pairs[b], c_w, dd)
        for b in range(_NB):
            for g in range(c_w // 16):
                plsc.store_scatter(pairs[b], [iota + g * 16, ones_i], ones_f)
        plsc.subcore_barrier()

        def build(k, b):
            for g in range(c_w // 16):
                vals = ew_t.at[k, pl.ds(g * 16, 16)][...]
                plsc.store_scatter(pairs[b], [iota + g * 16, zeros_i], vals)

        def fire(k, b):
            pltpu.async_copy(pairs[b], acc.at[dst_t.at[k]], ssem[b], add=True)

        def wait(b):
            pltpu.make_async_copy(pairs[b], acc.at[dst_t.at[0]],
                                  ssem[b]).wait()

        nj = (k_per + _NB - 1) // _NB

        @pl.loop(0, nj)
        def _(j):
            for b in range(_NB):
                k = j * _NB + b

                @pl.when(k < k_per)
                def _(k=k, b=b):
                    @pl.when(j > 0)
                    def _():
                        wait(b)
                    build(k, b)
                    fire(k, b)

        for b in range(min(_NB, k_per)):
            wait(b)
        plsc.subcore_barrier()
        pltpu.sync_copy(acc.at[pl.ds(s * rows_per_tile, rows_per_tile)],
                        out_hbm.at[c, pl.ds(s * rows_per_tile, rows_per_tile)])

    return pl.kernel(
        body,
        out_type=jax.ShapeDtypeStruct((_NC, n_acc, dd), jnp.float32),
        mesh=mesh,
        scratch_types=scratch,
        compiler_params=_sc_params(),
    )(dstr, ewr)


def _perm_matrix(d):
    perm = _unpack_perm(d)
    m = [[0.0] * d for _ in range(d)]
    for p, src in enumerate(perm):
        m[src][p] = 1.0
    return jnp.asarray(m, jnp.float32)


def _tc_matmul(x, w1, n, r):
    xd, hd = w1.shape

    def body(x_ref, w_ref, h_ref):
        h_ref[...] = jnp.dot(x_ref[...], w_ref[...],
                             preferred_element_type=jnp.float32)

    return pl.pallas_call(
        body,
        grid=(n // r,),
        in_specs=[
            pl.BlockSpec((r, xd), lambda i: (i, 0)),
            pl.BlockSpec((xd, hd), lambda i: (0, 0)),
        ],
        out_specs=pl.BlockSpec((r, hd), lambda i: (i, 0)),
        out_shape=jax.ShapeDtypeStruct((n, hd), jnp.float32),
    )(x, w1)


def _tc_prep(h, dacc, pm, n, r):
    hd = h.shape[1]
    h2 = hd // 2

    def body(d_ref, h_ref, p_ref, hs_ref, dv_ref, tab_ref):
        deg = d_ref[...][0, :, 0:2] + d_ref[...][1, :, 0:2] + 1.0
        dinv = lax.rsqrt(deg)
        hs = h_ref[...] * dinv[:, 0:1]
        hs_ref[0] = hs[:, :h2]
        hs_ref[1] = hs[:, h2:]
        dv_ref[...] = dinv
        tab_ref[0] = jnp.dot(hs[:, :h2], p_ref[...],
                             preferred_element_type=jnp.float32
                             ).astype(jnp.bfloat16)
        tab_ref[1] = jnp.dot(hs[:, h2:], p_ref[...],
                             preferred_element_type=jnp.float32
                             ).astype(jnp.bfloat16)

    return pl.pallas_call(
        body,
        grid=(n // r,),
        in_specs=[
            pl.BlockSpec((2, r, 16), lambda i: (0, i, 0)),
            pl.BlockSpec((r, hd), lambda i: (i, 0)),
            pl.BlockSpec((h2, h2), lambda i: (0, 0)),
        ],
        out_specs=[
            pl.BlockSpec((2, r, h2), lambda i: (0, i, 0)),
            pl.BlockSpec((r, 2), lambda i: (i, 0)),
            pl.BlockSpec((2, r, h2), lambda i: (0, i, 0)),
        ],
        out_shape=[
            jax.ShapeDtypeStruct((2, n, h2), jnp.float32),
            jax.ShapeDtypeStruct((n, 2), jnp.float32),
            jax.ShapeDtypeStruct((2, n, h2), jnp.bfloat16),
        ],
    )(dacc, h, pm)


def _tc_mid(p1, hs1s, dv, b1, wm, pm, n, r):
    hd = wm.shape[0]
    zd = wm.shape[1]
    h2 = hd // 2

    def body(p_ref, hs_ref, dv_ref, b_ref, w_ref, pm_ref, out_ref, tab_ref):
        d1 = dv_ref[...][:, 0:1]
        t_lo = (p_ref[0] + hs_ref[0]) * d1 + b_ref[...][:, :h2]
        t_hi = (p_ref[1] + hs_ref[1]) * d1 + b_ref[...][:, h2:]
        h_lo = jnp.maximum(t_lo, 0.0)
        h_hi = jnp.maximum(t_hi, 0.0)
        out = (jnp.dot(h_lo, w_ref[...][:h2, :],
                       preferred_element_type=jnp.float32)
               + jnp.dot(h_hi, w_ref[...][h2:, :],
                         preferred_element_type=jnp.float32))
        out = out * dv_ref[...][:, 1:2]
        out_ref[...] = out
        tab_ref[...] = jnp.dot(out, pm_ref[...],
                               preferred_element_type=jnp.float32
                               ).astype(jnp.bfloat16)

    return pl.pallas_call(
        body,
        grid=(n // r,),
        in_specs=[
            pl.BlockSpec((2, r, h2), lambda i: (0, i, 0)),
            pl.BlockSpec((2, r, h2), lambda i: (0, i, 0)),
            pl.BlockSpec((r, 2), lambda i: (i, 0)),
            pl.BlockSpec((1, hd), lambda i: (0, 0)),
            pl.BlockSpec((hd, zd), lambda i: (0, 0)),
            pl.BlockSpec((zd, zd), lambda i: (0, 0)),
        ],
        out_specs=[
            pl.BlockSpec((r, zd), lambda i: (i, 0)),
            pl.BlockSpec((r, zd), lambda i: (i, 0)),
        ],
        out_shape=[
            jax.ShapeDtypeStruct((n, zd), jnp.float32),
            jax.ShapeDtypeStruct((n, zd), jnp.bfloat16),
        ],
    )(p1, hs1s, dv, b1, wm, pm)


def _tc_post(p2, hs2, dv, bm, n, r):
    zd = hs2.shape[1]

    def body(p_ref, hs_ref, dv_ref, b_ref, out_ref):
        d2 = dv_ref[...][:, 1:2]
        out_ref[...] = (p_ref[0] + p_ref[1] + hs_ref[...]) * d2 + b_ref[...]

    return pl.pallas_call(
        body,
        grid=(n // r,),
        in_specs=[
            pl.BlockSpec((2, r, zd), lambda i: (0, i, 0)),
            pl.BlockSpec((r, zd), lambda i: (i, 0)),
            pl.BlockSpec((r, 2), lambda i: (i, 0)),
            pl.BlockSpec((1, zd), lambda i: (0, 0)),
        ],
        out_specs=pl.BlockSpec((r, zd), lambda i: (i, 0)),
        out_shape=jax.ShapeDtypeStruct((n, zd), jnp.float32),
    )(p2, hs2, dv, bm)


def kernel(x, ei, ew, W1, b1, Wm, bm):
    n, xd = x.shape
    hd = W1.shape[1]
    zd = Wm.shape[1]
    e = ei.shape[1]

    stride = _NC * _NS * 128 * 8
    ep = ((e + stride - 1) // stride) * stride
    pad = ep - e
    src = ei[0]
    dst = ei[1]
    ewp = ew.astype(jnp.float32)
    if pad:
        src = jnp.concatenate([src, jnp.zeros((pad,), src.dtype)])
        dst = jnp.concatenate([dst, jnp.full((pad,), n, dst.dtype)])
        ewp = jnp.concatenate([ewp, jnp.zeros((pad,), jnp.float32)])
    src64 = src.reshape(ep // 64, 64)
    dst64 = dst.reshape(ep // 64, 64)
    ew64 = ewp.reshape(ep // 64, 64)
    src128 = src.reshape(ep // 128, 128)
    dst128 = dst.reshape(ep // 128, 128)
    ew128 = ewp.reshape(ep // 128, 128)

    n_acc = ((n + 1 + _NS * 128 - 1) // (_NS * 128)) * (_NS * 128)
    r = 2000 if n % 2000 == 0 else n

    dacc = _sc_degrees(dst128, ew128, n_acc)
    h = _tc_matmul(x, W1, n, r)
    hs1s, dv, tab1 = _tc_prep(h, dacc, _perm_matrix(hd // 2), n, r)
    p1 = _sc_conv(tab1, src64, dst64, ew64, n_acc, feature_split=True)
    hs2, tab2 = _tc_mid(p1, hs1s, dv, b1.reshape(1, hd), Wm,
                        _perm_matrix(zd), n, r)
    p2 = _sc_conv(tab2, src128, dst128, None, n_acc, feature_split=False)
    mean = _tc_post(p2, hs2, dv, bm.reshape(1, zd), n, r)

    z = jnp.zeros((1,), jnp.float32)
    return (mean, z, z)
